# baseline (speedup 1.0000x reference)
import functools

import jax
import jax.numpy as jnp
from jax import lax
from jax.experimental import pallas as pl
from jax.experimental.pallas import tpu as pltpu
from jax.experimental.pallas import tpu_sc as plsc

BATCH = 16384
EMBED = 128
NC, NS = 2, 16
NW = NC * NS
B_PER_W = BATCH // NW
CHUNK = 64
NCHUNK = B_PER_W // CHUNK


def _sc_gather(users, items, t_ug, t_ig, t_um, t_im, pwv):
    mesh = plsc.VectorSubcoreMesh(core_axis_name="c", subcore_axis_name="s")
    row_t = jax.ShapeDtypeStruct((BATCH, EMBED), jnp.float32)
    score_t = jax.ShapeDtypeStruct((BATCH,), jnp.float32)

    @functools.partial(
        pl.kernel,
        mesh=mesh,
        out_type=(score_t, row_t, row_t),
        compiler_params=pltpu.CompilerParams(needs_layout_passes=False),
        scratch_types=[
            pltpu.VMEM((B_PER_W,), jnp.int32),
            pltpu.VMEM((B_PER_W,), jnp.int32),
            pltpu.VMEM((EMBED,), jnp.float32),
            pltpu.VMEM((B_PER_W,), jnp.float32),
        ] + [pltpu.VMEM((2, CHUNK, EMBED), jnp.float32)] * 4
          + [pltpu.SemaphoreType.DMA] * 12,
    )
    def k(u_ref, i_ref, ug_ref, ig_ref, um_ref, im_ref, pw_ref,
          o_sc, o_um, o_im, uidx, iidx, pwg, score_v,
          ugb, igb, umb, imb, *sems):
        gsem = sems[:8]
        ssem = sems[8:]
        wid = lax.axis_index("s") * NC + lax.axis_index("c")
        base = wid * B_PER_W
        pltpu.sync_copy(u_ref.at[pl.ds(base, B_PER_W)], uidx)
        pltpu.sync_copy(i_ref.at[pl.ds(base, B_PER_W)], iidx)
        pltpu.sync_copy(pw_ref.at[pl.ds(0, EMBED)], pwg)

        def fire(j, s):
            usl = uidx.at[pl.ds(j * CHUNK, CHUNK)]
            isl = iidx.at[pl.ds(j * CHUNK, CHUNK)]
            return (
                pltpu.async_copy(ug_ref.at[usl], ugb.at[s], gsem[4 * s + 0]),
                pltpu.async_copy(ig_ref.at[isl], igb.at[s], gsem[4 * s + 1]),
                pltpu.async_copy(um_ref.at[usl], umb.at[s], gsem[4 * s + 2]),
                pltpu.async_copy(im_ref.at[isl], imb.at[s], gsem[4 * s + 3]),
            )

        lanes = lax.iota(jnp.int32, 16)
        gh = [None, None]
        sh = [None, None]
        gh[0] = fire(0, 0)
        for j in range(NCHUNK):
            s = j & 1
            if j + 1 < NCHUNK:
                if sh[1 - s] is not None:
                    sh[1 - s][0].wait()
                    sh[1 - s][1].wait()
                gh[1 - s] = fire(j + 1, 1 - s)
            gh[s][0].wait()
            gh[s][1].wait()
            ngroups = CHUNK // 16
            rows_g = [lanes + 16 * g for g in range(ngroups)]

            def body(ci, accs):
                for cc in range(4):
                    c = ci * 4 + cc
                    cols = jnp.full((16,), 0, jnp.int32) + c
                    pwc = plsc.load_gather(pwg, [cols])
                    accs = tuple(
                        accs[g] + pwc
                        * plsc.load_gather(ugb.at[s], [rows_g[g], cols])
                        * plsc.load_gather(igb.at[s], [rows_g[g], cols])
                        for g in range(ngroups))
                return accs

            zero = jnp.zeros((16,), jnp.float32)
            accs = lax.fori_loop(0, EMBED // 4, body, (zero,) * ngroups)
            for g in range(ngroups):
                score_v[pl.ds(j * CHUNK + g * 16, 16)] = accs[g]
            gh[s][2].wait()
            gh[s][3].wait()
            sh[s] = (
                pltpu.async_copy(
                    umb.at[s], o_um.at[pl.ds(base + j * CHUNK, CHUNK)],
                    ssem[2 * s + 0]),
                pltpu.async_copy(
                    imb.at[s], o_im.at[pl.ds(base + j * CHUNK, CHUNK)],
                    ssem[2 * s + 1]),
            )
        for s in range(2):
            sh[s][0].wait()
            sh[s][1].wait()
        pltpu.sync_copy(score_v, o_sc.at[pl.ds(base, B_PER_W)])

    return k(users, items, t_ug, t_ig, t_um, t_im, pwv)


BLK = 1024


def _dense_body(gp, um, im, w1, b1, w2, b2, pwm, out):
    h = jnp.maximum(
        um[:] @ w1[0:EMBED, :] + im[:] @ w1[EMBED:2 * EMBED, :] + b1[:], 0.0)
    m = jnp.maximum(h @ w2[:] + b2[:], 0.0)
    out[:] = gp[:] + jnp.sum(m * pwm[:], axis=1)


def _tc_dense(gp, um, im, W1, b1, W2, b2, pwm):
    grid = (BATCH // BLK,)
    row_spec = pl.BlockSpec((BLK, EMBED), lambda i: (i, 0))
    full = lambda shape: pl.BlockSpec(shape, lambda i: (0,) * len(shape))
    return pl.pallas_call(
        _dense_body,
        grid=grid,
        in_specs=[
            pl.BlockSpec((BLK,), lambda i: (i,)),
            row_spec, row_spec,
            full((2 * EMBED, EMBED)), full((1, EMBED)),
            full((EMBED, 64)), full((1, 64)), full((1, 64)),
        ],
        out_specs=pl.BlockSpec((BLK,), lambda i: (i,)),
        out_shape=jax.ShapeDtypeStruct((BATCH,), jnp.float32),
    )(gp, um, im, W1, b1.reshape(1, EMBED), W2, b2.reshape(1, 64), pwm)


def kernel(users, items, user_emb_gmf, item_emb_gmf, user_emb_mlp,
           item_emb_mlp, W1, b1, W2, b2, proj_w):
    pwv = proj_w.reshape(EMBED + 64)
    gp, um, im = _sc_gather(users.astype(jnp.int32), items.astype(jnp.int32),
                            user_emb_gmf, item_emb_gmf, user_emb_mlp,
                            item_emb_mlp, pwv)
    pwm = pwv[EMBED:].reshape(1, 64)
    return _tc_dense(gp, um, im, W1, b1, W2, b2, pwm)


# speedup vs baseline: 1.8207x; 1.8207x over previous
import functools

import jax
import jax.numpy as jnp
from jax import lax
from jax.experimental import pallas as pl
from jax.experimental.pallas import tpu as pltpu
from jax.experimental.pallas import tpu_sc as plsc

BATCH = 16384
EMBED = 128
NC, NS = 2, 16
NW = NC * NS
CHUNK = 128
NSPLIT = 2
NBUF = 4


def _sc_gather4(users, items, t_ug, t_ig, t_um, t_im):
    n = users.shape[0]
    b_per_w = n // NW
    nchunk = b_per_w // CHUNK
    ntask = 4 * nchunk
    mesh = plsc.VectorSubcoreMesh(core_axis_name="c", subcore_axis_name="s")
    row_t = jax.ShapeDtypeStruct((n, EMBED), jnp.float32)

    @functools.partial(
        pl.kernel,
        mesh=mesh,
        out_type=(row_t, row_t, row_t, row_t),
        scratch_types=[
            pltpu.VMEM((b_per_w,), jnp.int32),
            pltpu.VMEM((b_per_w,), jnp.int32),
            pltpu.VMEM((NBUF, CHUNK, EMBED), jnp.float32),
        ] + [pltpu.SemaphoreType.DMA] * (2 * NBUF),
    )
    def k(u_ref, i_ref, ug_ref, ig_ref, um_ref, im_ref,
          o_ug, o_ig, o_um, o_im, uidx, iidx, buf, *sems):
        gsem, ssem = sems[:NBUF], sems[NBUF:]
        wid = lax.axis_index("s") * NC + lax.axis_index("c")
        base = wid * b_per_w
        pltpu.sync_copy(u_ref.at[pl.ds(base, b_per_w)], uidx)
        pltpu.sync_copy(i_ref.at[pl.ds(base, b_per_w)], iidx)
        tabs = (ug_ref, ig_ref, um_ref, im_ref)
        idxs = (uidx, iidx, uidx, iidx)
        outs = (o_ug, o_ig, o_um, o_im)
        tasks = [(tabs[t], idxs[t], outs[t], j)
                 for t in range(4) for j in range(nchunk)]

        def start_gather(kk):
            tb, ix, _, j = tasks[kk]
            b = kk % NBUF
            return pltpu.async_copy(
                tb.at[ix.at[pl.ds(j * CHUNK, CHUNK)]], buf.at[b], gsem[b])

        gh = [start_gather(b) for b in range(min(NBUF, ntask))]
        sh = [None] * NBUF
        for kk in range(ntask):
            b = kk % NBUF
            gh[b].wait()
            _, _, out, j = tasks[kk]
            sh[b] = pltpu.async_copy(
                buf.at[b], out.at[pl.ds(base + j * CHUNK, CHUNK)], ssem[b])
            if kk + NBUF < ntask:
                sh[b].wait()
                gh[b] = start_gather(kk + NBUF)
        for kk in range(max(ntask - NBUF, 0), ntask):
            sh[kk % NBUF].wait()

    return k(users, items, t_ug, t_ig, t_um, t_im)


BLK = 1024


def _dense_body(ug, ig, um, im, w1, b1, w2, b2, pwg, pwm, out):
    h = jnp.maximum(
        um[:] @ w1[0:EMBED, :] + im[:] @ w1[EMBED:2 * EMBED, :] + b1[:], 0.0)
    m = jnp.maximum(h @ w2[:] + b2[:], 0.0)
    out[:] = (jnp.sum(ug[:] * ig[:] * pwg[:], axis=1)
              + jnp.sum(m * pwm[:], axis=1))


def _tc_dense(ug, ig, um, im, W1, b1, W2, b2, pwg, pwm):
    n = ug.shape[0]
    grid = (n // BLK,)
    row_spec = pl.BlockSpec((BLK, EMBED), lambda i: (i, 0))
    full = lambda shape: pl.BlockSpec(shape, lambda i: (0,) * len(shape))
    return pl.pallas_call(
        _dense_body,
        grid=grid,
        in_specs=[
            row_spec, row_spec, row_spec, row_spec,
            full((2 * EMBED, EMBED)), full((1, EMBED)),
            full((EMBED, 64)), full((1, 64)),
            full((1, EMBED)), full((1, 64)),
        ],
        out_specs=pl.BlockSpec((BLK,), lambda i: (i,)),
        out_shape=jax.ShapeDtypeStruct((n,), jnp.float32),
    )(ug, ig, um, im, W1, b1, W2, b2, pwg, pwm)


def kernel(users, items, user_emb_gmf, item_emb_gmf, user_emb_mlp,
           item_emb_mlp, W1, b1, W2, b2, proj_w):
    users = users.astype(jnp.int32)
    items = items.astype(jnp.int32)
    b1r = b1.reshape(1, EMBED)
    b2r = b2.reshape(1, 64)
    pwg = proj_w[:EMBED].reshape(1, EMBED)
    pwm = proj_w[EMBED:].reshape(1, 64)
    n = BATCH // NSPLIT
    scores = []
    for si in range(NSPLIT):
        u_s = lax.slice_in_dim(users, si * n, (si + 1) * n)
        i_s = lax.slice_in_dim(items, si * n, (si + 1) * n)
        ug, ig, um, im = _sc_gather4(u_s, i_s, user_emb_gmf, item_emb_gmf,
                                     user_emb_mlp, item_emb_mlp)
        scores.append(_tc_dense(ug, ig, um, im, W1, b1r, W2, b2r, pwg, pwm))
    return jnp.concatenate(scores)
